# Initial kernel scaffold; baseline (speedup 1.0000x reference)
import jax
import jax.numpy as jnp
from jax.experimental import pallas as pl


def kernel(Gu, Gi, edge_index, user, pos):
    raise NotImplementedError("write your pallas kernel here")


# speedup vs baseline: 26.7033x; 26.7033x over previous
import functools

import jax
import jax.numpy as jnp
from jax import lax
from jax.experimental import pallas as pl
from jax.experimental.pallas import tpu as pltpu
from jax.experimental.pallas import tpu_sc as plsc

NU = 25000
NI = 25000
N = NU + NI
NPAD = 51200
D = 32
E = 1600000
NLAYERS = 3
BQ = 16384

NC, NS = 2, 16
NW = NC * NS
EPW = E // NW
CH = 400
NCHUNK = EPW // CH
RPW = NPAD // NS
QPW = BQ // NW

_MESH = plsc.VectorSubcoreMesh(core_axis_name="c", subcore_axis_name="s")
_SC_PARAMS = pltpu.CompilerParams(use_tc_tiling_on_sc=False)


def _wid():
    return lax.axis_index("s") * NC + lax.axis_index("c")


@functools.partial(
    pl.kernel,
    out_type=jax.ShapeDtypeStruct((NC * NPAD,), jnp.float32),
    mesh=_MESH,
    compiler_params=_SC_PARAMS,
    scratch_types=[
        pltpu.VMEM_SHARED((NPAD,), jnp.float32),
        pltpu.VMEM((CH,), jnp.int32),
        pltpu.VMEM((CH,), jnp.float32),
        pltpu.SemaphoreType.DMA,
    ],
)
def _k_deg(dst_hbm, ones_hbm, zeros1_hbm, degp_hbm, deg_sh, dst_v, ones_v, sem):
    cid = lax.axis_index("c")
    sid = lax.axis_index("s")
    wid = _wid()
    pltpu.sync_copy(zeros1_hbm.at[pl.ds(sid * RPW, RPW)],
                    deg_sh.at[pl.ds(sid * RPW, RPW)])
    pltpu.sync_copy(ones_hbm, ones_v)
    plsc.subcore_barrier()

    def body(i, _):
        base = wid * EPW + i * CH
        pltpu.sync_copy(dst_hbm.at[pl.ds(base, CH)], dst_v)
        pltpu.sync_copy(ones_v, deg_sh.at[dst_v], add=True)
        return 0

    lax.fori_loop(0, NCHUNK, body, 0)
    plsc.subcore_barrier()
    pltpu.sync_copy(deg_sh.at[pl.ds(sid * RPW, RPW)],
                    degp_hbm.at[pl.ds(cid * NPAD + sid * RPW, RPW)])


@functools.partial(
    pl.kernel,
    out_type=jax.ShapeDtypeStruct((NC, NPAD, D), jnp.float32),
    mesh=_MESH,
    compiler_params=_SC_PARAMS,
    scratch_types=[
        pltpu.VMEM_SHARED((NPAD, D), jnp.float32),
        pltpu.VMEM((CH,), jnp.int32),
        pltpu.VMEM((CH,), jnp.int32),
        pltpu.VMEM((CH, D), jnp.float32),
        pltpu.SemaphoreType.DMA,
    ],
)
def _k_layer(y_hbm, src_hbm, dst_hbm, zeros2_hbm, part_hbm,
             acc_sh, src_v, dst_v, rows_v, sem):
    cid = lax.axis_index("c")
    sid = lax.axis_index("s")
    wid = _wid()
    pltpu.sync_copy(zeros2_hbm.at[pl.ds(sid * RPW, RPW)],
                    acc_sh.at[pl.ds(sid * RPW, RPW)])
    plsc.subcore_barrier()

    def body(i, _):
        base = wid * EPW + i * CH
        pltpu.sync_copy(src_hbm.at[pl.ds(base, CH)], src_v)
        pltpu.sync_copy(dst_hbm.at[pl.ds(base, CH)], dst_v)
        pltpu.async_copy(y_hbm.at[src_v], rows_v, sem).wait()
        pltpu.sync_copy(rows_v, acc_sh.at[dst_v], add=True)
        return 0

    lax.fori_loop(0, NCHUNK, body, 0)
    plsc.subcore_barrier()
    pltpu.sync_copy(acc_sh.at[pl.ds(sid * RPW, RPW)],
                    part_hbm.at[cid, pl.ds(sid * RPW, RPW)])


@functools.partial(
    pl.kernel,
    out_type=[
        jax.ShapeDtypeStruct((BQ, D), jnp.float32),
        jax.ShapeDtypeStruct((BQ, D), jnp.float32),
    ],
    mesh=_MESH,
    compiler_params=_SC_PARAMS,
    scratch_types=[
        pltpu.VMEM((QPW,), jnp.int32),
        pltpu.VMEM((QPW,), jnp.int32),
        pltpu.VMEM((QPW, D), jnp.float32),
        pltpu.VMEM((QPW, D), jnp.float32),
        pltpu.SemaphoreType.DMA,
    ],
)
def _k_gather(s_hbm, user_hbm, pos_hbm, ug_hbm, pg_hbm,
              iu_v, ip_v, urows_v, prows_v, sem):
    wid = _wid()
    base = wid * QPW
    pltpu.sync_copy(user_hbm.at[pl.ds(base, QPW)], iu_v)
    pltpu.sync_copy(pos_hbm.at[pl.ds(base, QPW)], ip_v)

    def shift(j, _):
        off = pl.multiple_of(j * 16, 8)
        ip_v[pl.ds(off, 16)] = ip_v[pl.ds(off, 16)] + NU
        return 0

    lax.fori_loop(0, QPW // 16, shift, 0)

    pltpu.async_copy(s_hbm.at[iu_v], urows_v, sem).wait()
    pltpu.async_copy(s_hbm.at[ip_v], prows_v, sem).wait()
    pltpu.sync_copy(urows_v, ug_hbm.at[pl.ds(base, QPW)])
    pltpu.sync_copy(prows_v, pg_hbm.at[pl.ds(base, QPW)])


_ROWS = 2000
_GRID = N // _ROWS


def _k2_body(x0_ref, d0_ref, d1_ref, dinv_ref, y0_ref):
    deg = d0_ref[...] + d1_ref[...]
    dinv = jnp.where(deg > 0, lax.rsqrt(jnp.maximum(deg, 1.0)), 0.0)
    dinv_ref[...] = dinv
    y0_ref[...] = x0_ref[...] * dinv


def _k4_body(a0_ref, a1_ref, dinv_ref, s_ref, s_out_ref, y_ref):
    dinv = dinv_ref[...]
    x = (a0_ref[...] + a1_ref[...]) * dinv
    s_out_ref[...] = s_ref[...] + x
    y_ref[...] = x * dinv


def _k4_last_body(a0_ref, a1_ref, dinv_ref, s_ref, s_out_ref):
    x = (a0_ref[...] + a1_ref[...]) * dinv_ref[...]
    s_out_ref[...] = s_ref[...] + x


def _k_dot_body(u_ref, p_ref, o_ref):
    o_ref[...] = jnp.sum(u_ref[...] * p_ref[...], axis=1, keepdims=True) * (
        1.0 / 16.0)


_bs_rows = pl.BlockSpec((_ROWS, D), lambda i: (i, 0))
_bs_col = pl.BlockSpec((_ROWS, 1), lambda i: (i, 0))

_k2 = pl.pallas_call(
    _k2_body,
    grid=(_GRID,),
    in_specs=[_bs_rows, _bs_col, _bs_col],
    out_specs=[_bs_col, _bs_rows],
    out_shape=[
        jax.ShapeDtypeStruct((N, 1), jnp.float32),
        jax.ShapeDtypeStruct((N, D), jnp.float32),
    ],
)

_k4 = pl.pallas_call(
    _k4_body,
    grid=(_GRID,),
    in_specs=[_bs_rows, _bs_rows, _bs_col, _bs_rows],
    out_specs=[_bs_rows, _bs_rows],
    out_shape=[
        jax.ShapeDtypeStruct((N, D), jnp.float32),
        jax.ShapeDtypeStruct((N, D), jnp.float32),
    ],
)

_k4_last = pl.pallas_call(
    _k4_last_body,
    grid=(_GRID,),
    in_specs=[_bs_rows, _bs_rows, _bs_col, _bs_rows],
    out_specs=_bs_rows,
    out_shape=jax.ShapeDtypeStruct((N, D), jnp.float32),
)

_QROWS = 2048
_qs_rows = pl.BlockSpec((_QROWS, D), lambda i: (i, 0))
_qs_col = pl.BlockSpec((_QROWS, 1), lambda i: (i, 0))

_k_dot_tc = pl.pallas_call(
    _k_dot_body,
    grid=(BQ // _QROWS,),
    in_specs=[_qs_rows, _qs_rows],
    out_specs=_qs_col,
    out_shape=jax.ShapeDtypeStruct((BQ, 1), jnp.float32),
)


def kernel(Gu, Gi, edge_index, user, pos):
    src = edge_index[0]
    dst = edge_index[1]
    x0 = jnp.concatenate([Gu, Gi], axis=0)
    ones = jnp.ones((CH,), jnp.float32)
    zeros1 = jnp.zeros((NPAD,), jnp.float32)
    zeros2 = jnp.zeros((NPAD, D), jnp.float32)

    degp = _k_deg(dst, ones, zeros1).reshape(NC, NPAD)
    d0 = degp[0, :N, None]
    d1 = degp[1, :N, None]
    dinv, y = _k2(x0, d0, d1)

    s = x0
    for layer in range(NLAYERS):
        part = _k_layer(y, src, dst, zeros2)
        a0 = part[0, :N]
        a1 = part[1, :N]
        if layer < NLAYERS - 1:
            s, y = _k4(a0, a1, dinv, s)
        else:
            s = _k4_last(a0, a1, dinv, s)

    ug, pg = _k_gather(s, user, pos)
    return _k_dot_tc(ug, pg).reshape(BQ)
